# Initial kernel scaffold; baseline (speedup 1.0000x reference)
import jax
import jax.numpy as jnp
from jax.experimental import pallas as pl


def kernel(node_attr, edge_attr, edge_index, params):
    raise NotImplementedError("write your pallas kernel here")


# speedup vs baseline: 2.6584x; 2.6584x over previous
import functools

import jax
import jax.numpy as jnp
import numpy as np
from jax import lax
from jax.experimental import pallas as pl
from jax.experimental.pallas import tpu as pltpu
from jax.experimental.pallas import tpu_sc as plsc

N_NODES = 10000
N_EDGES = 160000
HID = 128

NC = 2
NS = 16
NW = NC * NS
CHUNK = 128
E_PAD = 163840
PER_W = E_PAD // NW
N_CHUNKS = PER_W // CHUNK
R = 10240
ROWS_PER_SUB = R // NS

BE = 2048
BN = 1024

_f32 = jnp.float32


def _dot(a, b):
    return jnp.dot(a.astype(jnp.bfloat16), b.astype(jnp.bfloat16),
                   preferred_element_type=_f32)


def _mlp_tail(h1, w2, b2, w3, b3, g, beta):
    h2 = jnp.maximum(_dot(h1, w2) + b2, 0.0)
    h3 = _dot(h2, w3) + b3
    mu = jnp.mean(h3, axis=-1, keepdims=True)
    var = jnp.mean((h3 - mu) ** 2, axis=-1, keepdims=True)
    return (h3 - mu) * lax.rsqrt(var + 1e-5) * g + beta


def _node_enc_body(x_ref, w1_ref, b1_ref, w2_ref, b2_ref, w3_ref, b3_ref,
                   g_ref, bt_ref, out_ref):
    h1 = jnp.maximum(_dot(x_ref[...], w1_ref[...]) + b1_ref[...], 0.0)
    out_ref[...] = _mlp_tail(h1, w2_ref[...], b2_ref[...], w3_ref[...],
                             b3_ref[...], g_ref[...], bt_ref[...])


def _edge_enc_body(x_ref, w1_ref, b1_ref, w2_ref, b2_ref, w3_ref, b3_ref,
                   g_ref, bt_ref, out_ref):
    h1 = jnp.maximum(_dot(x_ref[...], w1_ref[...]) + b1_ref[...], 0.0)
    out_ref[...] = _mlp_tail(h1, w2_ref[...], b2_ref[...], w3_ref[...],
                             b3_ref[...], g_ref[...], bt_ref[...])


def _edge_layer_body(gs_ref, gd_ref, eh_ref, w1s_ref, w1d_ref, w1e_ref,
                     b1_ref, w2_ref, b2_ref, w3_ref, b3_ref, g_ref, bt_ref,
                     out_ref):
    eh = eh_ref[...]
    h1 = jnp.maximum(
        _dot(gs_ref[...], w1s_ref[...]) + _dot(gd_ref[...], w1d_ref[...])
        + _dot(eh, w1e_ref[...]) + b1_ref[...], 0.0)
    out_ref[...] = eh + _mlp_tail(h1, w2_ref[...], b2_ref[...], w3_ref[...],
                                  b3_ref[...], g_ref[...], bt_ref[...])


def _node_layer_body(nh_ref, pa_ref, w1n_ref, w1a_ref, b1_ref, w2_ref,
                     b2_ref, w3_ref, b3_ref, g_ref, bt_ref, out_ref):
    nh = nh_ref[...]
    agg = pa_ref[0] + pa_ref[1]
    h1 = jnp.maximum(_dot(nh, w1n_ref[...]) + _dot(agg, w1a_ref[...])
                     + b1_ref[...], 0.0)
    out_ref[...] = nh + _mlp_tail(h1, w2_ref[...], b2_ref[...], w3_ref[...],
                                  b3_ref[...], g_ref[...], bt_ref[...])


def _dec_body(nh_ref, w1_ref, b1_ref, w2_ref, b2_ref, w3_ref, b3_ref,
              out_ref):
    h1 = jnp.maximum(_dot(nh_ref[...], w1_ref[...]) + b1_ref[...], 0.0)
    h2 = jnp.maximum(_dot(h1, w2_ref[...]) + b2_ref[...], 0.0)
    out_ref[...] = _dot(h2, w3_ref[...]) + b3_ref[...]


def _wspec(shape):
    nd = len(shape)
    return pl.BlockSpec(shape, lambda i: (0,) * nd)


def _rows(block, minor):
    return pl.BlockSpec((block, minor), lambda i: (i, 0))


def _tc_call(body, grid, in_specs, out_spec, out_shape, args):
    return pl.pallas_call(
        body,
        grid=grid,
        in_specs=in_specs,
        out_specs=out_spec,
        out_shape=out_shape,
    )(*args)


@functools.lru_cache(maxsize=None)
def _sc_gather_kernel():
    mesh = plsc.VectorSubcoreMesh(core_axis_name="c", subcore_axis_name="s")

    @functools.partial(
        pl.kernel,
        mesh=mesh,
        out_type=[jax.ShapeDtypeStruct((E_PAD, HID), _f32),
                  jax.ShapeDtypeStruct((E_PAD, HID), _f32)],
        scratch_types=[pltpu.VMEM((CHUNK,), jnp.int32),
                       pltpu.VMEM((CHUNK,), jnp.int32),
                       pltpu.VMEM((CHUNK, HID), _f32),
                       pltpu.VMEM((CHUNK, HID), _f32),
                       pltpu.SemaphoreType.DMA,
                       pltpu.SemaphoreType.DMA],
    )
    def gather_k(nh_hbm, src_hbm, dst_hbm, gs_hbm, gd_hbm,
                 idxs_v, idxd_v, rows_v, rowd_v, sems, semd):
        wid = lax.axis_index("s") * NC + lax.axis_index("c")
        base = wid * PER_W

        def body(j, carry):
            off = pl.multiple_of(base + j * CHUNK, CHUNK)
            pltpu.sync_copy(src_hbm.at[pl.ds(off, CHUNK)], idxs_v)
            pltpu.async_copy(nh_hbm.at[idxs_v], rows_v, sems).wait()
            pltpu.sync_copy(rows_v, gs_hbm.at[pl.ds(off, CHUNK)])
            pltpu.sync_copy(dst_hbm.at[pl.ds(off, CHUNK)], idxd_v)
            pltpu.async_copy(nh_hbm.at[idxd_v], rowd_v, semd).wait()
            pltpu.sync_copy(rowd_v, gd_hbm.at[pl.ds(off, CHUNK)])
            return carry

        lax.fori_loop(0, N_CHUNKS, body, 0)

    return gather_k


@functools.lru_cache(maxsize=None)
def _sc_scatter_kernel():
    mesh = plsc.VectorSubcoreMesh(core_axis_name="c", subcore_axis_name="s")

    @functools.partial(
        pl.kernel,
        mesh=mesh,
        out_type=jax.ShapeDtypeStruct((NC, R, HID), _f32),
        scratch_types=[pltpu.VMEM((1, CHUNK), jnp.int32),
                       pltpu.VMEM((CHUNK, HID), _f32),
                       pltpu.VMEM_SHARED((R, HID), _f32),
                       pltpu.SemaphoreType.DMA],
    )
    def scatter_k(enew_hbm, dst_hbm, zeros_hbm, out_hbm,
                  idx_v, rows_v, acc_sh, sem):
        cid = lax.axis_index("c")
        sid = lax.axis_index("s")
        wid = sid * NC + cid
        rbase = pl.multiple_of(sid * ROWS_PER_SUB, 8)
        pltpu.sync_copy(zeros_hbm.at[pl.ds(rbase, ROWS_PER_SUB)],
                        acc_sh.at[pl.ds(rbase, ROWS_PER_SUB)])
        plsc.subcore_barrier()
        base = wid * PER_W

        def body(j, carry):
            off = pl.multiple_of(base + j * CHUNK, CHUNK)
            pltpu.sync_copy(dst_hbm.at[pl.ds(off, CHUNK)], idx_v.at[0])
            pltpu.sync_copy(enew_hbm.at[pl.ds(off, CHUNK)], rows_v)
            pltpu.sync_copy(rows_v, acc_sh.at[idx_v.at[0]], add=True)
            return carry

        lax.fori_loop(0, N_CHUNKS, body, 0)
        plsc.subcore_barrier()
        pltpu.sync_copy(acc_sh.at[pl.ds(rbase, ROWS_PER_SUB)],
                        out_hbm.at[cid, pl.ds(rbase, ROWS_PER_SUB)])

    return scatter_k


def _sc_gather(nh, srcp, dstp):
    return _sc_gather_kernel()(nh, srcp, dstp)


def _sc_scatter(e_new, dstp, zeros_r):
    return _sc_scatter_kernel()(e_new, dstp, zeros_r)


def _fourier_features(pos):
    sp = pos[:, :2]
    fi = jnp.arange(-3, 4, dtype=pos.dtype)
    freqs = (2.0 ** fi) * jnp.pi
    se = sp[:, :, None] * freqs[None, None, :]
    ff = jnp.concatenate([jnp.cos(se), jnp.sin(se)], axis=-1)
    return ff.reshape(pos.shape[0], -1)


def _mlp_params(p):
    (w1, b1), (w2, b2), (w3, b3) = p["layers"]
    out = [w1, b1.reshape(1, -1), w2, b2.reshape(1, -1), w3, b3.reshape(1, -1)]
    if p["ln"] is not None:
        g, beta = p["ln"]
        out += [g.reshape(1, -1), beta.reshape(1, -1)]
    return out


def kernel(node_attr, edge_attr, edge_index, params):
    src = edge_index[0]
    dst = edge_index[1]
    pad_e = E_PAD - N_EDGES
    ar = jnp.arange(pad_e, dtype=jnp.int32)
    srcp = jnp.concatenate([src, ar % N_NODES])
    dstp = jnp.concatenate([dst, N_NODES + ar % (R - N_NODES)])
    x_enc = jnp.concatenate([node_attr, _fourier_features(node_attr)], axis=-1)
    x_pad = jnp.pad(x_enc, ((0, R - N_NODES), (0, 0)))
    ea_pad = jnp.pad(edge_attr, ((0, pad_e), (0, 0)))
    zeros_r = jnp.zeros((R, HID), _f32)

    w1, b1, w2, b2, w3, b3, g, bt = _mlp_params(params["node_encoder"])
    nh = _tc_call(
        _node_enc_body, (R // BN,),
        [_rows(BN, 40), _wspec((40, HID)), _wspec((1, HID)),
         _wspec((HID, HID)), _wspec((1, HID)), _wspec((HID, HID)),
         _wspec((1, HID)), _wspec((1, HID)), _wspec((1, HID))],
        _rows(BN, HID), jax.ShapeDtypeStruct((R, HID), _f32),
        (x_pad, w1, b1, w2, b2, w3, b3, g, bt))

    w1, b1, w2, b2, w3, b3, g, bt = _mlp_params(params["edge_encoder"])
    eh = _tc_call(
        _edge_enc_body, (E_PAD // BE,),
        [_rows(BE, 4), _wspec((4, HID)), _wspec((1, HID)),
         _wspec((HID, HID)), _wspec((1, HID)), _wspec((HID, HID)),
         _wspec((1, HID)), _wspec((1, HID)), _wspec((1, HID))],
        _rows(BE, HID), jax.ShapeDtypeStruct((E_PAD, HID), _f32),
        (ea_pad, w1, b1, w2, b2, w3, b3, g, bt))

    for layer in params["layers"]:
        gs, gd = _sc_gather(nh, srcp, dstp)

        w1, b1, w2, b2, w3, b3, g, bt = _mlp_params(layer["edge_mlp"])
        w1s, w1d, w1e = w1[:HID], w1[HID:2 * HID], w1[2 * HID:]
        e_new = _tc_call(
            _edge_layer_body, (E_PAD // BE,),
            [_rows(BE, HID)] * 3
            + [_wspec((HID, HID)), _wspec((HID, HID)), _wspec((HID, HID)),
               _wspec((1, HID)), _wspec((HID, HID)), _wspec((1, HID)),
               _wspec((HID, HID)), _wspec((1, HID)), _wspec((1, HID)),
               _wspec((1, HID))],
            _rows(BE, HID), jax.ShapeDtypeStruct((E_PAD, HID), _f32),
            (gs, gd, eh, w1s, w1d, w1e, b1, w2, b2, w3, b3, g, bt))

        partials = _sc_scatter(e_new, dstp, zeros_r)

        w1, b1, w2, b2, w3, b3, g, bt = _mlp_params(layer["node_mlp"])
        w1n, w1a = w1[:HID], w1[HID:]
        nh = _tc_call(
            _node_layer_body, (R // BN,),
            [_rows(BN, HID),
             pl.BlockSpec((NC, BN, HID), lambda i: (0, i, 0)),
             _wspec((HID, HID)), _wspec((HID, HID)), _wspec((1, HID)),
             _wspec((HID, HID)), _wspec((1, HID)), _wspec((HID, HID)),
             _wspec((1, HID)), _wspec((1, HID)), _wspec((1, HID))],
            _rows(BN, HID), jax.ShapeDtypeStruct((R, HID), _f32),
            (nh, partials, w1n, w1a, b1, w2, b2, w3, b3, g, bt))
        eh = e_new

    w1, b1, w2, b2, w3, b3 = _mlp_params(params["decoder"])
    out = _tc_call(
        _dec_body, (R // BN,),
        [_rows(BN, HID), _wspec((HID, HID)), _wspec((1, HID)),
         _wspec((HID, HID)), _wspec((1, HID)), _wspec((HID, 3)),
         _wspec((1, 3))],
        _rows(BN, 3), jax.ShapeDtypeStruct((R, 3), _f32),
        (nh, w1, b1, w2, b2, w3, b3))
    return out[:N_NODES]


# speedup vs baseline: 3.5146x; 1.3221x over previous
import functools

import jax
import jax.numpy as jnp
import numpy as np
from jax import lax
from jax.experimental import pallas as pl
from jax.experimental.pallas import tpu as pltpu
from jax.experimental.pallas import tpu_sc as plsc

N_NODES = 10000
N_EDGES = 160000
HID = 128

NC = 2
NS = 16
NW = NC * NS
CHUNK = 128
E_PAD = 163840
PER_W = E_PAD // NW
N_CHUNKS = PER_W // CHUNK
R = 10240
ROWS_PER_SUB = R // NS

BE = 2048
BN = 1024

_f32 = jnp.float32


def _dot(a, b):
    return jnp.dot(a.astype(jnp.bfloat16), b.astype(jnp.bfloat16),
                   preferred_element_type=_f32)


def _mlp_tail(h1, w2, b2, w3, b3, g, beta):
    h2 = jnp.maximum(_dot(h1, w2) + b2, 0.0)
    h3 = _dot(h2, w3) + b3
    mu = jnp.mean(h3, axis=-1, keepdims=True)
    var = jnp.mean((h3 - mu) ** 2, axis=-1, keepdims=True)
    return (h3 - mu) * lax.rsqrt(var + 1e-5) * g + beta


def _node_enc_body(x_ref, w1_ref, b1_ref, w2_ref, b2_ref, w3_ref, b3_ref,
                   g_ref, bt_ref, out_ref):
    h1 = jnp.maximum(_dot(x_ref[...], w1_ref[...]) + b1_ref[...], 0.0)
    out_ref[...] = _mlp_tail(h1, w2_ref[...], b2_ref[...], w3_ref[...],
                             b3_ref[...], g_ref[...], bt_ref[...])


def _edge_enc_body(x_ref, w1_ref, b1_ref, w2_ref, b2_ref, w3_ref, b3_ref,
                   g_ref, bt_ref, out_ref):
    h1 = jnp.maximum(_dot(x_ref[...], w1_ref[...]) + b1_ref[...], 0.0)
    out_ref[...] = _mlp_tail(h1, w2_ref[...], b2_ref[...], w3_ref[...],
                             b3_ref[...], g_ref[...], bt_ref[...])


def _edge_layer_body(gs_ref, gd_ref, eh_ref, w1s_ref, w1d_ref, w1e_ref,
                     b1_ref, w2_ref, b2_ref, w3_ref, b3_ref, g_ref, bt_ref,
                     out_ref):
    eh = eh_ref[...]
    h1 = jnp.maximum(
        _dot(gs_ref[...], w1s_ref[...]) + _dot(gd_ref[...], w1d_ref[...])
        + _dot(eh, w1e_ref[...]) + b1_ref[...], 0.0)
    out_ref[...] = eh + _mlp_tail(h1, w2_ref[...], b2_ref[...], w3_ref[...],
                                  b3_ref[...], g_ref[...], bt_ref[...])


def _node_layer_body(nh_ref, pa_ref, w1n_ref, w1a_ref, b1_ref, w2_ref,
                     b2_ref, w3_ref, b3_ref, g_ref, bt_ref, out_ref):
    nh = nh_ref[...]
    agg = pa_ref[0] + pa_ref[1]
    h1 = jnp.maximum(_dot(nh, w1n_ref[...]) + _dot(agg, w1a_ref[...])
                     + b1_ref[...], 0.0)
    out_ref[...] = nh + _mlp_tail(h1, w2_ref[...], b2_ref[...], w3_ref[...],
                                  b3_ref[...], g_ref[...], bt_ref[...])


def _dec_body(nh_ref, w1_ref, b1_ref, w2_ref, b2_ref, w3_ref, b3_ref,
              out_ref):
    h1 = jnp.maximum(_dot(nh_ref[...], w1_ref[...]) + b1_ref[...], 0.0)
    h2 = jnp.maximum(_dot(h1, w2_ref[...]) + b2_ref[...], 0.0)
    out_ref[...] = _dot(h2, w3_ref[...]) + b3_ref[...]


def _wspec(shape):
    nd = len(shape)
    return pl.BlockSpec(shape, lambda i: (0,) * nd)


def _rows(block, minor):
    return pl.BlockSpec((block, minor), lambda i: (i, 0))


def _tc_call(body, grid, in_specs, out_spec, out_shape, args):
    return pl.pallas_call(
        body,
        grid=grid,
        in_specs=in_specs,
        out_specs=out_spec,
        out_shape=out_shape,
    )(*args)


@functools.lru_cache(maxsize=None)
def _sc_gather_kernel():
    mesh = plsc.VectorSubcoreMesh(core_axis_name="c", subcore_axis_name="s")

    @functools.partial(
        pl.kernel,
        mesh=mesh,
        out_type=[jax.ShapeDtypeStruct((E_PAD, HID), _f32),
                  jax.ShapeDtypeStruct((E_PAD, HID), _f32)],
        scratch_types=[pltpu.VMEM((N_CHUNKS, CHUNK), jnp.int32),
                       pltpu.VMEM((N_CHUNKS, CHUNK), jnp.int32)]
                      + [pltpu.VMEM((CHUNK, HID), _f32)] * 4
                      + [pltpu.SemaphoreType.DMA] * 8,
    )
    def gather_k(nh_hbm, src_hbm, dst_hbm, gs_hbm, gd_hbm,
                 idxs_v, idxd_v, sa, sb, da, db,
                 gsa, gda, gsb, gdb, wsa, wda, wsb, wdb):
        wid = lax.axis_index("s") * NC + lax.axis_index("c")
        base = wid * PER_W
        pltpu.sync_copy(src_hbm.at[wid], idxs_v)
        pltpu.sync_copy(dst_hbm.at[wid], idxd_v)

        def g_start(j, bufs, bufd, sems, semd):
            pltpu.async_copy(nh_hbm.at[idxs_v.at[j]], bufs, sems)
            pltpu.async_copy(nh_hbm.at[idxd_v.at[j]], bufd, semd)

        def g_wait(bufs, bufd, sems, semd):
            pltpu.make_async_copy(nh_hbm.at[pl.ds(0, CHUNK)], bufs, sems).wait()
            pltpu.make_async_copy(nh_hbm.at[pl.ds(0, CHUNK)], bufd, semd).wait()

        def w_start(j, bufs, bufd, sems, semd):
            off = pl.multiple_of(base + j * CHUNK, CHUNK)
            pltpu.async_copy(bufs, gs_hbm.at[pl.ds(off, CHUNK)], sems)
            pltpu.async_copy(bufd, gd_hbm.at[pl.ds(off, CHUNK)], semd)

        def w_wait(bufs, bufd, sems, semd):
            pltpu.make_async_copy(bufs, gs_hbm.at[pl.ds(base, CHUNK)], sems).wait()
            pltpu.make_async_copy(bufd, gd_hbm.at[pl.ds(base, CHUNK)], semd).wait()

        g_start(0, sa, da, gsa, gda)
        g_start(1, sb, db, gsb, gdb)

        def body(k, carry):
            j0 = 2 * k
            g_wait(sa, da, gsa, gda)
            w_start(j0, sa, da, wsa, wda)
            g_wait(sb, db, gsb, gdb)
            w_start(j0 + 1, sb, db, wsb, wdb)
            w_wait(sa, da, wsa, wda)
            g_start(lax.rem(j0 + 2, N_CHUNKS), sa, da, gsa, gda)
            w_wait(sb, db, wsb, wdb)
            g_start(lax.rem(j0 + 3, N_CHUNKS), sb, db, gsb, gdb)
            return carry

        lax.fori_loop(0, N_CHUNKS // 2, body, 0)
        g_wait(sa, da, gsa, gda)
        g_wait(sb, db, gsb, gdb)

    return gather_k


@functools.lru_cache(maxsize=None)
def _sc_scatter_kernel():
    mesh = plsc.VectorSubcoreMesh(core_axis_name="c", subcore_axis_name="s")

    @functools.partial(
        pl.kernel,
        mesh=mesh,
        out_type=jax.ShapeDtypeStruct((NC, R, HID), _f32),
        scratch_types=[pltpu.VMEM((N_CHUNKS, CHUNK), jnp.int32),
                       pltpu.VMEM((CHUNK, HID), _f32),
                       pltpu.VMEM((CHUNK, HID), _f32),
                       pltpu.VMEM_SHARED((R, HID), _f32),
                       pltpu.SemaphoreType.DMA,
                       pltpu.SemaphoreType.DMA],
    )
    def scatter_k(enew_hbm, dst_hbm, zeros_hbm, out_hbm,
                  idx_v, ra, rb, acc_sh, la, lb):
        cid = lax.axis_index("c")
        sid = lax.axis_index("s")
        wid = sid * NC + cid
        rbase = pl.multiple_of(sid * ROWS_PER_SUB, 8)
        pltpu.sync_copy(zeros_hbm.at[pl.ds(rbase, ROWS_PER_SUB)],
                        acc_sh.at[pl.ds(rbase, ROWS_PER_SUB)])
        pltpu.sync_copy(dst_hbm.at[wid], idx_v)
        plsc.subcore_barrier()
        base = wid * PER_W

        def l_start(j, buf, sem):
            off = pl.multiple_of(base + j * CHUNK, CHUNK)
            pltpu.async_copy(enew_hbm.at[pl.ds(off, CHUNK)], buf, sem)

        def l_wait(buf, sem):
            pltpu.make_async_copy(enew_hbm.at[pl.ds(0, CHUNK)], buf, sem).wait()

        l_start(0, ra, la)
        l_start(1, rb, lb)

        def body(k, carry):
            j0 = 2 * k
            l_wait(ra, la)
            pltpu.sync_copy(ra, acc_sh.at[idx_v.at[j0]], add=True)
            l_start(lax.rem(j0 + 2, N_CHUNKS), ra, la)
            l_wait(rb, lb)
            pltpu.sync_copy(rb, acc_sh.at[idx_v.at[j0 + 1]], add=True)
            l_start(lax.rem(j0 + 3, N_CHUNKS), rb, lb)
            return carry

        lax.fori_loop(0, N_CHUNKS // 2, body, 0)
        l_wait(ra, la)
        l_wait(rb, lb)
        plsc.subcore_barrier()
        pltpu.sync_copy(acc_sh.at[pl.ds(rbase, ROWS_PER_SUB)],
                        out_hbm.at[cid, pl.ds(rbase, ROWS_PER_SUB)])

    return scatter_k


def _sc_gather(nh, srcp, dstp):
    src3 = srcp.reshape(NW, N_CHUNKS, CHUNK)
    dst3 = dstp.reshape(NW, N_CHUNKS, CHUNK)
    return _sc_gather_kernel()(nh, src3, dst3)


def _sc_scatter(e_new, dstp, zeros_r):
    dst3 = dstp.reshape(NW, N_CHUNKS, CHUNK)
    return _sc_scatter_kernel()(e_new, dst3, zeros_r)


def _fourier_features(pos):
    sp = pos[:, :2]
    fi = jnp.arange(-3, 4, dtype=pos.dtype)
    freqs = (2.0 ** fi) * jnp.pi
    se = sp[:, :, None] * freqs[None, None, :]
    ff = jnp.concatenate([jnp.cos(se), jnp.sin(se)], axis=-1)
    return ff.reshape(pos.shape[0], -1)


def _mlp_params(p):
    (w1, b1), (w2, b2), (w3, b3) = p["layers"]
    out = [w1, b1.reshape(1, -1), w2, b2.reshape(1, -1), w3, b3.reshape(1, -1)]
    if p["ln"] is not None:
        g, beta = p["ln"]
        out += [g.reshape(1, -1), beta.reshape(1, -1)]
    return out


def kernel(node_attr, edge_attr, edge_index, params):
    src = edge_index[0]
    dst = edge_index[1]
    pad_e = E_PAD - N_EDGES
    ar = jnp.arange(pad_e, dtype=jnp.int32)
    srcp = jnp.concatenate([src, ar % N_NODES])
    dstp = jnp.concatenate([dst, N_NODES + ar % (R - N_NODES)])
    x_enc = jnp.concatenate([node_attr, _fourier_features(node_attr)], axis=-1)
    x_pad = jnp.pad(x_enc, ((0, R - N_NODES), (0, 0)))
    ea_pad = jnp.pad(edge_attr, ((0, pad_e), (0, 0)))
    zeros_r = jnp.zeros((R, HID), _f32)

    w1, b1, w2, b2, w3, b3, g, bt = _mlp_params(params["node_encoder"])
    nh = _tc_call(
        _node_enc_body, (R // BN,),
        [_rows(BN, 40), _wspec((40, HID)), _wspec((1, HID)),
         _wspec((HID, HID)), _wspec((1, HID)), _wspec((HID, HID)),
         _wspec((1, HID)), _wspec((1, HID)), _wspec((1, HID))],
        _rows(BN, HID), jax.ShapeDtypeStruct((R, HID), _f32),
        (x_pad, w1, b1, w2, b2, w3, b3, g, bt))

    w1, b1, w2, b2, w3, b3, g, bt = _mlp_params(params["edge_encoder"])
    eh = _tc_call(
        _edge_enc_body, (E_PAD // BE,),
        [_rows(BE, 4), _wspec((4, HID)), _wspec((1, HID)),
         _wspec((HID, HID)), _wspec((1, HID)), _wspec((HID, HID)),
         _wspec((1, HID)), _wspec((1, HID)), _wspec((1, HID))],
        _rows(BE, HID), jax.ShapeDtypeStruct((E_PAD, HID), _f32),
        (ea_pad, w1, b1, w2, b2, w3, b3, g, bt))

    for layer in params["layers"]:
        gs, gd = _sc_gather(nh, srcp, dstp)

        w1, b1, w2, b2, w3, b3, g, bt = _mlp_params(layer["edge_mlp"])
        w1s, w1d, w1e = w1[:HID], w1[HID:2 * HID], w1[2 * HID:]
        e_new = _tc_call(
            _edge_layer_body, (E_PAD // BE,),
            [_rows(BE, HID)] * 3
            + [_wspec((HID, HID)), _wspec((HID, HID)), _wspec((HID, HID)),
               _wspec((1, HID)), _wspec((HID, HID)), _wspec((1, HID)),
               _wspec((HID, HID)), _wspec((1, HID)), _wspec((1, HID)),
               _wspec((1, HID))],
            _rows(BE, HID), jax.ShapeDtypeStruct((E_PAD, HID), _f32),
            (gs, gd, eh, w1s, w1d, w1e, b1, w2, b2, w3, b3, g, bt))

        partials = _sc_scatter(e_new, dstp, zeros_r)

        w1, b1, w2, b2, w3, b3, g, bt = _mlp_params(layer["node_mlp"])
        w1n, w1a = w1[:HID], w1[HID:]
        nh = _tc_call(
            _node_layer_body, (R // BN,),
            [_rows(BN, HID),
             pl.BlockSpec((NC, BN, HID), lambda i: (0, i, 0)),
             _wspec((HID, HID)), _wspec((HID, HID)), _wspec((1, HID)),
             _wspec((HID, HID)), _wspec((1, HID)), _wspec((HID, HID)),
             _wspec((1, HID)), _wspec((1, HID)), _wspec((1, HID))],
            _rows(BN, HID), jax.ShapeDtypeStruct((R, HID), _f32),
            (nh, partials, w1n, w1a, b1, w2, b2, w3, b3, g, bt))
        eh = e_new

    w1, b1, w2, b2, w3, b3 = _mlp_params(params["decoder"])
    out = _tc_call(
        _dec_body, (R // BN,),
        [_rows(BN, HID), _wspec((HID, HID)), _wspec((1, HID)),
         _wspec((HID, HID)), _wspec((1, HID)), _wspec((HID, 3)),
         _wspec((1, 3))],
        _rows(BN, 3), jax.ShapeDtypeStruct((R, 3), _f32),
        (nh, w1, b1, w2, b2, w3, b3))
    return out[:N_NODES]
